# baseline (speedup 1.0000x reference)
import functools

import numpy as np
import jax
import jax.numpy as jnp
from jax.experimental import pallas as pl
from jax.experimental.pallas import tpu as pltpu

_NT = 10
_COLS = 1024
_CH = 16
_FLUSH = 8

_THR32 = np.linspace(0.0, 1.0, _NT + 2, dtype=np.float32)[1:-1]
_LOGITS = tuple(
    float(np.log(t / (1.0 - t))) for t in _THR32.astype(np.float64)
)


def _fold16(x):
    s = x[:, 0:128]
    for j in range(1, _COLS // 128):
        s = s + x[:, j * 128:(j + 1) * 128]
    return s


def _prec_rec_kernel(pred_ref, mask_ref, targ_ref, out_ref, acc16_ref,
                     acc32_ref, *, nb):
    i = pl.program_id(1)

    @pl.when(i == 0)
    def _init():
        acc16_ref[...] = jnp.zeros_like(acc16_ref)
        acc32_ref[...] = jnp.zeros_like(acc32_ref)

    n_ch = pred_ref.shape[0] // _CH

    def _flush():
        for k in range(_NT + 1):
            a16 = acc16_ref[k]
            for half, v32 in enumerate(
                    (a16[0:8, :].astype(jnp.int32),
                     a16[8:16, :].astype(jnp.int32))):
                contrib = (v32 & 255) + jnp.left_shift(
                    jnp.right_shift(v32, 8), 16)
                acc32_ref[k, half * 8:(half + 1) * 8, :] += contrib
            acc16_ref[k] = jnp.zeros_like(a16)

    for c in range(n_ch):
        sl = slice(c * _CH, (c + 1) * _CH)
        p16 = pred_ref[sl, :].astype(jnp.bfloat16)
        m = mask_ref[sl, :]
        t = targ_ref[sl, :]
        w16 = (jnp.left_shift(m & t, 8) + m).astype(jnp.int16)
        acc16_ref[_NT] += _fold16(w16)
        for k in range(_NT):
            sel = jnp.where(p16 > _LOGITS[k], w16, jnp.int16(0))
            acc16_ref[k] += _fold16(sel)
        if (c + 1) % _FLUSH == 0:
            _flush()

    @pl.when(i == nb - 1)
    def _finalize():
        for k in range(_NT + 1):
            a = acc32_ref[k]
            hi = jnp.right_shift(a, 16)
            lo = a & 0xFFFF
            out_ref[0, 0, 2 * k] = jnp.sum(hi).astype(jnp.float32)
            out_ref[0, 0, 2 * k + 1] = jnp.sum(lo).astype(jnp.float32)


def kernel(prediction, mask, target):
    rows = prediction.size // _COLS
    pred2 = prediction.reshape(rows, _COLS)
    mask2 = mask.reshape(rows, _COLS)
    targ2 = target.reshape(rows, _COLS)

    blk_r = 1024
    nb = rows // (2 * blk_r)

    in_spec = pl.BlockSpec((blk_r, _COLS), lambda c, i: (c * nb + i, 0))
    out = pl.pallas_call(
        functools.partial(_prec_rec_kernel, nb=nb),
        out_shape=jax.ShapeDtypeStruct((2, 1, 2 * (_NT + 1)), jnp.float32),
        grid=(2, nb),
        in_specs=[in_spec, in_spec, in_spec],
        out_specs=pl.BlockSpec(
            (1, 1, 2 * (_NT + 1)), lambda c, i: (c, 0, 0),
            memory_space=pltpu.SMEM),
        scratch_shapes=[
            pltpu.VMEM((_NT + 1, 16, 128), jnp.int16),
            pltpu.VMEM((_NT + 1, 16, 128), jnp.int32),
        ],
        compiler_params=pltpu.CompilerParams(
            dimension_semantics=("parallel", "arbitrary"),
        ),
        name="prec_rec",
    )(pred2, mask2, targ2)

    c = out[0, 0] + out[1, 0]
    tp = c[0:2 * _NT:2]
    p = c[1:2 * _NT:2]
    total_t = c[2 * _NT]
    total_m = c[2 * _NT + 1]
    fp = p - tp
    fn = total_t - tp
    tn = total_m - p - fn
    return jnp.stack([tp, fp, tn, fn], axis=0)


# speedup vs baseline: 1.0314x; 1.0314x over previous
import functools

import numpy as np
import jax
import jax.numpy as jnp
from jax.experimental import pallas as pl
from jax.experimental.pallas import tpu as pltpu

_NT = 10
_COLS = 1024
_CH = 16
_FLUSH = 8

_THR32 = np.linspace(0.0, 1.0, _NT + 2, dtype=np.float32)[1:-1]
_LOGITS = tuple(
    float(np.log(t / (1.0 - t))) for t in _THR32.astype(np.float64)
)


def _fold16(x):
    s = x[:, 0:128]
    for j in range(1, _COLS // 128):
        s = s + x[:, j * 128:(j + 1) * 128]
    return s


def _prec_rec_kernel(pred_ref, mask_ref, targ_ref, out_ref, acc16_ref,
                     acc32_ref, *, nb):
    i = pl.program_id(0)

    @pl.when(i == 0)
    def _init():
        acc16_ref[...] = jnp.zeros_like(acc16_ref)
        acc32_ref[...] = jnp.zeros_like(acc32_ref)

    n_ch = pred_ref.shape[0] // _CH

    def _flush():
        for k in range(_NT + 1):
            a16 = acc16_ref[k]
            for half, v32 in enumerate(
                    (a16[0:8, :].astype(jnp.int32),
                     a16[8:16, :].astype(jnp.int32))):
                contrib = (v32 & 255) + jnp.left_shift(
                    jnp.right_shift(v32, 8), 16)
                acc32_ref[k, half * 8:(half + 1) * 8, :] += contrib
            acc16_ref[k] = jnp.zeros_like(a16)

    for c in range(n_ch):
        sl = slice(c * _CH, (c + 1) * _CH)
        p16 = pred_ref[sl, :].astype(jnp.bfloat16)
        m = mask_ref[sl, :]
        t = targ_ref[sl, :]
        w16 = (jnp.left_shift(m & t, 8) + m).astype(jnp.int16)
        acc16_ref[_NT] += _fold16(w16)
        for k in range(_NT):
            sel = jnp.where(p16 > _LOGITS[k], w16, jnp.int16(0))
            acc16_ref[k] += _fold16(sel)
        if (c + 1) % _FLUSH == 0:
            _flush()

    @pl.when(i == nb - 1)
    def _finalize():
        for k in range(_NT + 1):
            a = acc32_ref[k]
            hi = jnp.right_shift(a, 16)
            lo = a & 0xFFFF
            out_ref[0, 2 * k] = jnp.sum(hi).astype(jnp.float32)
            out_ref[0, 2 * k + 1] = jnp.sum(lo).astype(jnp.float32)


def kernel(prediction, mask, target):
    rows = prediction.size // _COLS
    pred2 = prediction.reshape(rows, _COLS)
    mask2 = mask.reshape(rows, _COLS)
    targ2 = target.reshape(rows, _COLS)

    blk_r = 2048
    nb = rows // blk_r

    in_spec = pl.BlockSpec((blk_r, _COLS), lambda i: (i, 0))
    out = pl.pallas_call(
        functools.partial(_prec_rec_kernel, nb=nb),
        out_shape=jax.ShapeDtypeStruct((1, 2 * (_NT + 1)), jnp.float32),
        grid=(nb,),
        in_specs=[in_spec, in_spec, in_spec],
        out_specs=pl.BlockSpec(
            (1, 2 * (_NT + 1)), lambda i: (0, 0),
            memory_space=pltpu.SMEM),
        scratch_shapes=[
            pltpu.VMEM((_NT + 1, 16, 128), jnp.int16),
            pltpu.VMEM((_NT + 1, 16, 128), jnp.int32),
        ],
        compiler_params=pltpu.CompilerParams(
            dimension_semantics=("arbitrary",),
        ),
        name="prec_rec",
    )(pred2, mask2, targ2)

    c = out[0]
    tp = c[0:2 * _NT:2]
    p = c[1:2 * _NT:2]
    total_t = c[2 * _NT]
    total_m = c[2 * _NT + 1]
    fp = p - tp
    fn = total_t - tp
    tn = total_m - p - fn
    return jnp.stack([tp, fp, tn, fn], axis=0)
